# baseline (speedup 1.0000x reference)
import functools

import numpy as np
import jax
from jax import lax
import jax.numpy as jnp
from jax.experimental import pallas as pl
from jax.experimental.pallas import tpu as pltpu
from jax.experimental.pallas import tpu_sc as plsc

_SF = 2
_NUM_B = 64

_K_FALLBACK = [2, -2, -2, 1, 2, 1, 2, -2, 2, 1, -2, 2, 1, 1, 1, -2,
               -1, 1, 2, -1, 2, -1, -2, -2, 1, 2, 0, 2, 1, 0, 1, 2,
               2, 1, -1, 1, 2, 0, 1, 1, -2, -2, 0, 1, -2, -1, 1, 2,
               0, 0, 1, -2, -1, 2, 2, -2, -2, -2, -1, 2, -1, 0, 1, 2]


def _static_shifts():
    try:
        k_key = jax.random.fold_in(jax.random.key(0), 1234)
        ks = jax.random.randint(k_key, (_NUM_B,), -_SF, _SF + 1)
        return [int(v) for v in np.asarray(ks)]
    except Exception:
        return list(_K_FALLBACK)


_K_LIST = _static_shifts()

_NUM_WORKERS = 32
_PAD = 8


def kernel(x):
    B, C, N = x.shape
    mesh = plsc.VectorSubcoreMesh(core_axis_name="c", subcore_axis_name="s")

    @functools.partial(
        pl.kernel,
        out_type=jax.ShapeDtypeStruct(x.shape, x.dtype),
        mesh=mesh,
        scratch_types=[
            pltpu.VMEM((N + 2 * _PAD,), jnp.float32),
            pltpu.VMEM((N,), jnp.float32),
            pltpu.SemaphoreType.DMA,
        ],
        compiler_params=pltpu.CompilerParams(
            use_tc_tiling_on_sc=False, needs_layout_passes=False
        ),
    )
    def phase_shuffle(x_hbm, o_hbm, inb, outb, sem):
        wid = lax.axis_index("s") * 2 + lax.axis_index("c")

        def process_batch(b, k):
            iota16 = lax.iota(jnp.int32, 16)
            front_idx = _PAD + jnp.abs(iota16 - k)
            jr = iota16 + (N - 16 - k)
            back_idx = _PAD + jnp.minimum(jr, 2 * (N - 1) - jr)
            shift = _PAD - k

            @pl.loop(0, C)
            def _(c):
                pltpu.async_copy(
                    x_hbm.at[b, c, :], inb.at[pl.ds(_PAD, N)], sem
                ).wait()

                @pl.loop(0, N, step=16)
                def _(j0):
                    outb[pl.ds(j0, 16)] = inb[pl.ds(j0 + shift, 16)]

                outb[pl.ds(0, 16)] = plsc.load_gather(inb, [front_idx])
                outb[pl.ds(N - 16, 16)] = plsc.load_gather(inb, [back_idx])

                pltpu.async_copy(outb, o_hbm.at[b, c, :], sem).wait()

        for w in range(_NUM_WORKERS):

            @pl.when(wid == w)
            def _(w=w):
                for lb in range(B // _NUM_WORKERS):
                    b = (B // _NUM_WORKERS) * w + lb
                    process_batch(b, _K_LIST[b])

    return phase_shuffle(x)


# speedup vs baseline: 1.2557x; 1.2557x over previous
import functools

import numpy as np
import jax
from jax import lax
import jax.numpy as jnp
from jax.experimental import pallas as pl
from jax.experimental.pallas import tpu as pltpu
from jax.experimental.pallas import tpu_sc as plsc

_SF = 2
_NUM_B = 64

_K_FALLBACK = [2, -2, -2, 1, 2, 1, 2, -2, 2, 1, -2, 2, 1, 1, 1, -2,
               -1, 1, 2, -1, 2, -1, -2, -2, 1, 2, 0, 2, 1, 0, 1, 2,
               2, 1, -1, 1, 2, 0, 1, 1, -2, -2, 0, 1, -2, -1, 1, 2,
               0, 0, 1, -2, -1, 2, 2, -2, -2, -2, -1, 2, -1, 0, 1, 2]


def _static_shifts():
    try:
        k_key = jax.random.fold_in(jax.random.key(0), 1234)
        ks = jax.random.randint(k_key, (_NUM_B,), -_SF, _SF + 1)
        return [int(v) for v in np.asarray(ks)]
    except Exception:
        return list(_K_FALLBACK)


_K_LIST = _static_shifts()

_K_PACKED = [
    sum((_K_LIST[8 * w + j] + _SF) << (3 * j) for j in range(8))
    for w in range(_NUM_B // 8)
]

_NUM_WORKERS = 32
_PAD = 8
_UNROLL = 16


def kernel(x):
    B, C, N = x.shape
    rows = B * C
    rows_per_w = rows // _NUM_WORKERS
    log_c = C.bit_length() - 1
    mesh = plsc.VectorSubcoreMesh(core_axis_name="c", subcore_axis_name="s")

    @functools.partial(
        pl.kernel,
        out_type=jax.ShapeDtypeStruct(x.shape, x.dtype),
        mesh=mesh,
        scratch_types=[
            pltpu.VMEM((2, N + 2 * _PAD), jnp.float32),
            pltpu.VMEM((2, N), jnp.float32),
            pltpu.SemaphoreType.DMA,
            pltpu.SemaphoreType.DMA,
            pltpu.SemaphoreType.DMA,
            pltpu.SemaphoreType.DMA,
        ],
        compiler_params=pltpu.CompilerParams(
            use_tc_tiling_on_sc=False, needs_layout_passes=False
        ),
    )
    def phase_shuffle(x_hbm, o_hbm, inb, outb, si0, si1, so0, so1):
        wid = lax.axis_index("s") * 2 + lax.axis_index("c")
        base = wid * rows_per_w
        sem_in = (si0, si1)
        sem_out = (so0, so1)

        def row_bc(r):
            t = base + r
            return lax.shift_right_logical(t, log_c), lax.bitwise_and(t, C - 1)

        def shift_of(b):
            word = jnp.int32(_K_PACKED[0])
            idx = lax.shift_right_logical(b, 3)
            for w in range(1, len(_K_PACKED)):
                word = jnp.where(idx == w, jnp.int32(_K_PACKED[w]), word)
            field = 3 * lax.bitwise_and(b, 7)
            return (
                lax.bitwise_and(lax.shift_right_logical(word, field), 7) - _SF
            )

        def start_in(r, s):
            b, c = row_bc(r)
            pltpu.async_copy(
                x_hbm.at[b, c, :], inb.at[s, pl.ds(_PAD, N)], sem_in[s]
            )

        def wait_in(s):
            pltpu.make_async_copy(
                x_hbm.at[0, 0, :], inb.at[s, pl.ds(_PAD, N)], sem_in[s]
            ).wait()

        def start_out(r, s):
            b, c = row_bc(r)
            pltpu.async_copy(outb.at[s], o_hbm.at[b, c, :], sem_out[s])

        def wait_out(s):
            pltpu.make_async_copy(
                outb.at[s], o_hbm.at[0, 0, :], sem_out[s]
            ).wait()

        def compute(r, s):
            b, _ = row_bc(r)
            k = shift_of(b)
            shift = _PAD - k
            iota16 = lax.iota(jnp.int32, 16)

            @pl.loop(0, N, step=16 * _UNROLL)
            def _(j0):
                sb = j0 + shift
                for u in range(_UNROLL):
                    outb[s, pl.ds(j0 + 16 * u, 16)] = inb[
                        s, pl.ds(sb + 16 * u, 16)
                    ]

            front_idx = _PAD + jnp.abs(iota16 - k)
            jr = iota16 + ((N - 16) - k)
            back_idx = _PAD + jnp.minimum(jr, 2 * (N - 1) - jr)
            outb[s, pl.ds(0, 16)] = plsc.load_gather(inb.at[s], [front_idx])
            outb[s, pl.ds(N - 16, 16)] = plsc.load_gather(inb.at[s], [back_idx])

        start_in(0, 0)
        start_in(1, 1)

        @pl.loop(0, rows_per_w, step=2)
        def _(i):
            for s in (0, 1):
                r = i + s
                wait_in(s)

                @pl.when(r >= 2)
                def _():
                    wait_out(s)

                compute(r, s)
                start_out(r, s)

                @pl.when(r + 2 < rows_per_w)
                def _():
                    start_in(r + 2, s)

        wait_out(0)
        wait_out(1)

    return phase_shuffle(x)


# speedup vs baseline: 1.9172x; 1.5268x over previous
import functools

import numpy as np
import jax
from jax import lax
import jax.numpy as jnp
from jax.experimental import pallas as pl
from jax.experimental.pallas import tpu as pltpu
from jax.experimental.pallas import tpu_sc as plsc

_SF = 2
_NUM_B = 64

_K_FALLBACK = [2, -2, -2, 1, 2, 1, 2, -2, 2, 1, -2, 2, 1, 1, 1, -2,
               -1, 1, 2, -1, 2, -1, -2, -2, 1, 2, 0, 2, 1, 0, 1, 2,
               2, 1, -1, 1, 2, 0, 1, 1, -2, -2, 0, 1, -2, -1, 1, 2,
               0, 0, 1, -2, -1, 2, 2, -2, -2, -2, -1, 2, -1, 0, 1, 2]


def _static_shifts():
    try:
        k_key = jax.random.fold_in(jax.random.key(0), 1234)
        ks = jax.random.randint(k_key, (_NUM_B,), -_SF, _SF + 1)
        return [int(v) for v in np.asarray(ks)]
    except Exception:
        return list(_K_FALLBACK)


_K_LIST = _static_shifts()

_K_PACKED = [
    sum((_K_LIST[8 * w + j] + _SF) << (3 * j) for j in range(8))
    for w in range(_NUM_B // 8)
]

_NUM_WORKERS = 32
_PAD = 8
_UNROLL = 16


def kernel(x):
    B, C, N = x.shape
    rows = B * C
    rows_per_w = rows // _NUM_WORKERS
    log_c = C.bit_length() - 1
    mesh = plsc.VectorSubcoreMesh(core_axis_name="c", subcore_axis_name="s")

    @functools.partial(
        pl.kernel,
        out_type=jax.ShapeDtypeStruct(x.shape, x.dtype),
        mesh=mesh,
        scratch_types=[
            pltpu.VMEM((2, N + 2 * _PAD), jnp.float32),
            pltpu.VMEM((2, N), jnp.float32),
            pltpu.SemaphoreType.DMA,
            pltpu.SemaphoreType.DMA,
            pltpu.SemaphoreType.DMA,
            pltpu.SemaphoreType.DMA,
        ],
        compiler_params=pltpu.CompilerParams(
            use_tc_tiling_on_sc=False, needs_layout_passes=False
        ),
    )
    def phase_shuffle(x_hbm, o_hbm, inb, outb, si0, si1, so0, so1):
        wid = lax.axis_index("s") * 2 + lax.axis_index("c")
        base = wid * rows_per_w
        sem_in = (si0, si1)
        sem_out = (so0, so1)

        def row_bc(r):
            t = base + r
            return lax.shift_right_logical(t, log_c), lax.bitwise_and(t, C - 1)

        def shift_of(b):
            word = jnp.int32(_K_PACKED[0])
            idx = lax.shift_right_logical(b, 3)
            for w in range(1, len(_K_PACKED)):
                word = jnp.where(idx == w, jnp.int32(_K_PACKED[w]), word)
            field = 3 * lax.bitwise_and(b, 7)
            return (
                lax.bitwise_and(lax.shift_right_logical(word, field), 7) - _SF
            )

        def start_in(r, s):
            b, c = row_bc(r)
            pltpu.async_copy(
                x_hbm.at[b, c, :], inb.at[s, pl.ds(_PAD, N)], sem_in[s]
            )

        def wait_in(s):
            pltpu.make_async_copy(
                x_hbm.at[0, 0, :], inb.at[s, pl.ds(_PAD, N)], sem_in[s]
            ).wait()

        def start_out(r, s):
            b, c = row_bc(r)
            pltpu.async_copy(outb.at[s], o_hbm.at[b, c, :], sem_out[s])

        def wait_out(s):
            pltpu.make_async_copy(
                outb.at[s], o_hbm.at[0, 0, :], sem_out[s]
            ).wait()

        def compute(r, s):
            b, _ = row_bc(r)
            k = shift_of(b)
            shift = _PAD - k
            iota16 = lax.iota(jnp.int32, 16)

            @plsc.parallel_loop(0, N, step=16, unroll=_UNROLL)
            def _(j0):
                outb[s, pl.ds(j0, 16)] = inb[s, pl.ds(j0 + shift, 16)]

            front_idx = _PAD + jnp.abs(iota16 - k)
            jr = iota16 + ((N - 16) - k)
            back_idx = _PAD + jnp.minimum(jr, 2 * (N - 1) - jr)
            outb[s, pl.ds(0, 16)] = plsc.load_gather(inb.at[s], [front_idx])
            outb[s, pl.ds(N - 16, 16)] = plsc.load_gather(inb.at[s], [back_idx])

        start_in(0, 0)
        start_in(1, 1)

        @pl.loop(0, rows_per_w, step=2)
        def _(i):
            for s in (0, 1):
                r = i + s
                wait_in(s)

                @pl.when(r >= 2)
                def _():
                    wait_out(s)

                compute(r, s)
                start_out(r, s)

                @pl.when(r + 2 < rows_per_w)
                def _():
                    start_in(r + 2, s)

        wait_out(0)
        wait_out(1)

    return phase_shuffle(x)


# speedup vs baseline: 4.2930x; 2.2392x over previous
import functools

import numpy as np
import jax
from jax import lax
import jax.numpy as jnp
from jax.experimental import pallas as pl
from jax.experimental.pallas import tpu as pltpu
from jax.experimental.pallas import tpu_sc as plsc

_SF = 2
_NUM_B = 64

_K_FALLBACK = [2, -2, -2, 1, 2, 1, 2, -2, 2, 1, -2, 2, 1, 1, 1, -2,
               -1, 1, 2, -1, 2, -1, -2, -2, 1, 2, 0, 2, 1, 0, 1, 2,
               2, 1, -1, 1, 2, 0, 1, 1, -2, -2, 0, 1, -2, -1, 1, 2,
               0, 0, 1, -2, -1, 2, 2, -2, -2, -2, -1, 2, -1, 0, 1, 2]


def _static_shifts():
    try:
        k_key = jax.random.fold_in(jax.random.key(0), 1234)
        ks = jax.random.randint(k_key, (_NUM_B,), -_SF, _SF + 1)
        return [int(v) for v in np.asarray(ks)]
    except Exception:
        return list(_K_FALLBACK)


_K_LIST = _static_shifts()

_K_PACKED = [
    sum((_K_LIST[8 * w + j] + _SF) << (3 * j) for j in range(8))
    for w in range(_NUM_B // 8)
]

_NUM_WORKERS = 32
_W = 2048
_H = 128


def kernel(x):
    B, C, N = x.shape
    n_chunks = N // _W
    rgs = (B * C // 8) // _NUM_WORKERS
    tiles = _W // 128
    mesh = plsc.VectorSubcoreMesh(core_axis_name="c", subcore_axis_name="s")

    @functools.partial(
        pl.kernel,
        out_type=jax.ShapeDtypeStruct(x.shape, x.dtype),
        mesh=mesh,
        scratch_types=[
            pltpu.VMEM((2, 8, _W + 2 * _H), jnp.float32),
            pltpu.VMEM((2, 8, _W), jnp.float32),
            pltpu.SemaphoreType.DMA,
            pltpu.SemaphoreType.DMA,
            pltpu.SemaphoreType.DMA,
            pltpu.SemaphoreType.DMA,
        ],
        compiler_params=pltpu.CompilerParams(needs_layout_passes=False),
    )
    def phase_shuffle(x_hbm, o_hbm, inb, outb, si0, si1, so0, so1):
        wid = lax.axis_index("s") * 2 + lax.axis_index("c")
        sem_in = (si0, si1)
        sem_out = (so0, so1)

        def rg_bc(rg):
            b = 2 * wid + lax.shift_right_logical(rg, 3)
            c0 = pl.multiple_of(lax.bitwise_and(rg, 7) * 8, 8)
            return b, c0

        def shift_of(b):
            word = jnp.int32(_K_PACKED[0])
            idx = lax.shift_right_logical(b, 3)
            for w in range(1, len(_K_PACKED)):
                word = jnp.where(idx == w, jnp.int32(_K_PACKED[w]), word)
            field = 3 * lax.bitwise_and(b, 7)
            return (
                lax.bitwise_and(lax.shift_right_logical(word, field), 7) - _SF
            )

        def in_slices(q):
            j0 = q * _W
            if q == 0:
                return pl.ds(0, _W + _H), pl.ds(_H, _W + _H)
            if q == n_chunks - 1:
                return pl.ds(j0 - _H, _W + _H), pl.ds(0, _W + _H)
            return pl.ds(j0 - _H, _W + 2 * _H), pl.ds(0, _W + 2 * _H)

        def start_in(q, s, b, c0):
            src, dst = in_slices(q)
            pltpu.async_copy(
                x_hbm.at[b, pl.ds(c0, 8), src], inb.at[s, :, dst], sem_in[s]
            )

        def wait_in(q, s):
            src, dst = in_slices(q)
            pltpu.make_async_copy(
                x_hbm.at[0, pl.ds(0, 8), src], inb.at[s, :, dst], sem_in[s]
            ).wait()

        def start_out(q, s, b, c0):
            pltpu.async_copy(
                outb.at[s], o_hbm.at[b, pl.ds(c0, 8), pl.ds(q * _W, _W)],
                sem_out[s],
            )

        def wait_out(s):
            pltpu.make_async_copy(
                outb.at[s], o_hbm.at[0, pl.ds(0, 8), pl.ds(0, _W)], sem_out[s]
            ).wait()

        def compute(q, s, k):
            iota16 = lax.iota(jnp.int32, 16)
            ins = inb.at[s]
            outs = outb.at[s]
            ic = iota16 + (_H - k)
            oc = iota16

            @plsc.parallel_loop(0, 8 * tiles)
            def _(i):
                t = lax.shift_right_logical(i, 3)
                r = lax.bitwise_and(i, 7)
                rv = jnp.broadcast_to(r, (16,))
                tb = t * 128
                for w in range(8):
                    v = plsc.load_gather(ins, [rv, ic + (tb + 16 * w)])
                    plsc.store_scatter(outs, [rv, oc + (tb + 16 * w)], v)

            if q == 0:
                fv = _H + jnp.abs(iota16 - k)

                @plsc.parallel_loop(0, 8)
                def _(r):
                    rv = jnp.broadcast_to(r, (16,))
                    v = plsc.load_gather(ins, [rv, fv])
                    plsc.store_scatter(outs, [rv, oc], v)

            if q == n_chunks - 1:
                jr = iota16 + (_W - 16) - k
                bv = _H + jnp.minimum(jr, 2 * _W - 2 - jr)

                @plsc.parallel_loop(0, 8)
                def _(r):
                    rv = jnp.broadcast_to(r, (16,))
                    v = plsc.load_gather(ins, [rv, bv])
                    plsc.store_scatter(outs, [rv, oc + (_W - 16)], v)

        b0, c00 = rg_bc(0)
        start_in(0, 0, b0, c00)
        start_in(1, 1, b0, c00)

        @pl.loop(0, rgs)
        def _(rg):
            b, c0 = rg_bc(rg)
            k = shift_of(b)
            for q in range(n_chunks):
                s = q & 1
                wait_in(q, s)

                @pl.when(rg * n_chunks + q >= 2)
                def _():
                    wait_out(s)

                compute(q, s, k)
                start_out(q, s, b, c0)

                if q < n_chunks - 2:
                    start_in(q + 2, s, b, c0)
                else:

                    @pl.when(rg + 1 < rgs)
                    def _(q=q):
                        b2, c02 = rg_bc(rg + 1)
                        start_in(q + 2 - n_chunks, s, b2, c02)

        wait_out(0)
        wait_out(1)

    return phase_shuffle(x)


# speedup vs baseline: 5.4647x; 1.2729x over previous
import functools

import numpy as np
import jax
from jax import lax
import jax.numpy as jnp
from jax.experimental import pallas as pl
from jax.experimental.pallas import tpu as pltpu
from jax.experimental.pallas import tpu_sc as plsc

_SF = 2
_NUM_B = 64

_K_FALLBACK = [2, -2, -2, 1, 2, 1, 2, -2, 2, 1, -2, 2, 1, 1, 1, -2,
               -1, 1, 2, -1, 2, -1, -2, -2, 1, 2, 0, 2, 1, 0, 1, 2,
               2, 1, -1, 1, 2, 0, 1, 1, -2, -2, 0, 1, -2, -1, 1, 2,
               0, 0, 1, -2, -1, 2, 2, -2, -2, -2, -1, 2, -1, 0, 1, 2]


def _static_shifts():
    try:
        k_key = jax.random.fold_in(jax.random.key(0), 1234)
        ks = jax.random.randint(k_key, (_NUM_B,), -_SF, _SF + 1)
        return [int(v) for v in np.asarray(ks)]
    except Exception:
        return list(_K_FALLBACK)


_K_LIST = _static_shifts()

_K_PACKED = [
    sum((_K_LIST[8 * w + j] + _SF) << (3 * j) for j in range(8))
    for w in range(_NUM_B // 8)
]

_NUM_WORKERS = 32
_W = 2048
_H = 128


def kernel(x):
    B, C, N = x.shape
    n_chunks = N // _W
    rgs = (B * C // 8) // _NUM_WORKERS
    tiles = _W // 128
    mesh = plsc.VectorSubcoreMesh(core_axis_name="c", subcore_axis_name="s")

    @functools.partial(
        pl.kernel,
        out_type=jax.ShapeDtypeStruct(x.shape, x.dtype),
        mesh=mesh,
        scratch_types=[
            pltpu.VMEM((2, 8, _W + 2 * _H), jnp.float32),
            pltpu.VMEM((2, 8, _W), jnp.float32),
            pltpu.SemaphoreType.DMA,
            pltpu.SemaphoreType.DMA,
            pltpu.SemaphoreType.DMA,
            pltpu.SemaphoreType.DMA,
        ],
        compiler_params=pltpu.CompilerParams(needs_layout_passes=False),
    )
    def phase_shuffle(x_hbm, o_hbm, inb, outb, si0, si1, so0, so1):
        wid = lax.axis_index("s") * 2 + lax.axis_index("c")
        sem_in = (si0, si1)
        sem_out = (so0, so1)

        def rg_bc(rg):
            b = 2 * wid + lax.shift_right_logical(rg, 3)
            c0 = pl.multiple_of(lax.bitwise_and(rg, 7) * 8, 8)
            return b, c0

        def shift_of(b):
            word = jnp.int32(_K_PACKED[0])
            idx = lax.shift_right_logical(b, 3)
            for w in range(1, len(_K_PACKED)):
                word = jnp.where(idx == w, jnp.int32(_K_PACKED[w]), word)
            field = 3 * lax.bitwise_and(b, 7)
            return (
                lax.bitwise_and(lax.shift_right_logical(word, field), 7) - _SF
            )

        def in_slices(q):
            j0 = q * _W
            if q == 0:
                return pl.ds(0, _W + _H), pl.ds(_H, _W + _H)
            if q == n_chunks - 1:
                return pl.ds(j0 - _H, _W + _H), pl.ds(0, _W + _H)
            return pl.ds(j0 - _H, _W + 2 * _H), pl.ds(0, _W + 2 * _H)

        def start_in(q, s, b, c0):
            src, dst = in_slices(q)
            pltpu.async_copy(
                x_hbm.at[b, pl.ds(c0, 8), src], inb.at[s, :, dst], sem_in[s]
            )

        def wait_in(q, s):
            src, dst = in_slices(q)
            pltpu.make_async_copy(
                x_hbm.at[0, pl.ds(0, 8), src], inb.at[s, :, dst], sem_in[s]
            ).wait()

        def start_out(q, s, b, c0):
            pltpu.async_copy(
                outb.at[s], o_hbm.at[b, pl.ds(c0, 8), pl.ds(q * _W, _W)],
                sem_out[s],
            )

        def wait_out(s):
            pltpu.make_async_copy(
                outb.at[s], o_hbm.at[0, pl.ds(0, 8), pl.ds(0, _W)], sem_out[s]
            ).wait()

        def compute(q, s, k):
            iota16 = lax.iota(jnp.int32, 16)
            ins = inb.at[s]
            outs = outb.at[s]
            ic = iota16 + (_H - k)
            oc = iota16

            @plsc.parallel_loop(0, 8 * tiles, unroll=2)
            def _(i):
                t = lax.shift_right_logical(i, 3)
                r = lax.bitwise_and(i, 7)
                rv = jnp.broadcast_to(r, (16,))
                tb = t * 128
                for w in range(8):
                    v = plsc.load_gather(ins, [rv, ic + (tb + 16 * w)])
                    plsc.store_scatter(outs, [rv, oc + (tb + 16 * w)], v)

            if q == 0:
                fv = _H + jnp.abs(iota16 - k)

                @plsc.parallel_loop(0, 8)
                def _(r):
                    rv = jnp.broadcast_to(r, (16,))
                    v = plsc.load_gather(ins, [rv, fv])
                    plsc.store_scatter(outs, [rv, oc], v)

            if q == n_chunks - 1:
                jr = iota16 + (_W - 16) - k
                bv = _H + jnp.minimum(jr, 2 * _W - 2 - jr)

                @plsc.parallel_loop(0, 8)
                def _(r):
                    rv = jnp.broadcast_to(r, (16,))
                    v = plsc.load_gather(ins, [rv, bv])
                    plsc.store_scatter(outs, [rv, oc + (_W - 16)], v)

        b0, c00 = rg_bc(0)
        start_in(0, 0, b0, c00)
        start_in(1, 1, b0, c00)

        @pl.loop(0, rgs)
        def _(rg):
            b, c0 = rg_bc(rg)
            k = shift_of(b)
            for q in range(n_chunks):
                s = q & 1
                wait_in(q, s)

                @pl.when(rg * n_chunks + q >= 2)
                def _():
                    wait_out(s)

                compute(q, s, k)
                start_out(q, s, b, c0)

                if q < n_chunks - 2:
                    start_in(q + 2, s, b, c0)
                else:

                    @pl.when(rg + 1 < rgs)
                    def _(q=q):
                        b2, c02 = rg_bc(rg + 1)
                        start_in(q + 2 - n_chunks, s, b2, c02)

        wait_out(0)
        wait_out(1)

    return phase_shuffle(x)


# speedup vs baseline: 5.6381x; 1.0317x over previous
import functools

import numpy as np
import jax
from jax import lax
import jax.numpy as jnp
from jax.experimental import pallas as pl
from jax.experimental.pallas import tpu as pltpu
from jax.experimental.pallas import tpu_sc as plsc

_SF = 2
_NUM_B = 64

_K_FALLBACK = [2, -2, -2, 1, 2, 1, 2, -2, 2, 1, -2, 2, 1, 1, 1, -2,
               -1, 1, 2, -1, 2, -1, -2, -2, 1, 2, 0, 2, 1, 0, 1, 2,
               2, 1, -1, 1, 2, 0, 1, 1, -2, -2, 0, 1, -2, -1, 1, 2,
               0, 0, 1, -2, -1, 2, 2, -2, -2, -2, -1, 2, -1, 0, 1, 2]


def _static_shifts():
    try:
        k_key = jax.random.fold_in(jax.random.key(0), 1234)
        ks = jax.random.randint(k_key, (_NUM_B,), -_SF, _SF + 1)
        return [int(v) for v in np.asarray(ks)]
    except Exception:
        return list(_K_FALLBACK)


_K_LIST = _static_shifts()

_K_PACKED = [
    sum((_K_LIST[8 * w + j] + _SF) << (3 * j) for j in range(8))
    for w in range(_NUM_B // 8)
]

_NUM_WORKERS = 32
_W = 2048
_H = 128


def kernel(x):
    B, C, N = x.shape
    n_chunks = N // _W
    rgs = (B * C // 8) // _NUM_WORKERS
    tiles = _W // 128
    mesh = plsc.VectorSubcoreMesh(core_axis_name="c", subcore_axis_name="s")

    @functools.partial(
        pl.kernel,
        out_type=jax.ShapeDtypeStruct(x.shape, x.dtype),
        mesh=mesh,
        scratch_types=[
            pltpu.VMEM((2, 8, _W + 2 * _H), jnp.float32),
            pltpu.VMEM((2, 8, _W), jnp.float32),
            pltpu.SemaphoreType.DMA,
            pltpu.SemaphoreType.DMA,
            pltpu.SemaphoreType.DMA,
            pltpu.SemaphoreType.DMA,
        ],
        compiler_params=pltpu.CompilerParams(needs_layout_passes=False),
    )
    def phase_shuffle(x_hbm, o_hbm, inb, outb, si0, si1, so0, so1):
        wid = lax.axis_index("s") * 2 + lax.axis_index("c")
        sem_in = (si0, si1)
        sem_out = (so0, so1)

        def rg_bc(rg):
            b = 2 * wid + lax.shift_right_logical(rg, 3)
            c0 = pl.multiple_of(lax.bitwise_and(rg, 7) * 8, 8)
            return b, c0

        def shift_of(b):
            word = jnp.int32(_K_PACKED[0])
            idx = lax.shift_right_logical(b, 3)
            for w in range(1, len(_K_PACKED)):
                word = jnp.where(idx == w, jnp.int32(_K_PACKED[w]), word)
            field = 3 * lax.bitwise_and(b, 7)
            return (
                lax.bitwise_and(lax.shift_right_logical(word, field), 7) - _SF
            )

        def in_slices(q):
            j0 = q * _W
            if q == 0:
                return pl.ds(0, _W + _H), pl.ds(_H, _W + _H)
            if q == n_chunks - 1:
                return pl.ds(j0 - _H, _W + _H), pl.ds(0, _W + _H)
            return pl.ds(j0 - _H, _W + 2 * _H), pl.ds(0, _W + 2 * _H)

        def start_in(q, s, b, c0):
            src, dst = in_slices(q)
            pltpu.async_copy(
                x_hbm.at[b, pl.ds(c0, 8), src], inb.at[s, :, dst], sem_in[s]
            )

        def wait_in(q, s):
            src, dst = in_slices(q)
            pltpu.make_async_copy(
                x_hbm.at[0, pl.ds(0, 8), src], inb.at[s, :, dst], sem_in[s]
            ).wait()

        def start_out(q, s, b, c0):
            pltpu.async_copy(
                outb.at[s], o_hbm.at[b, pl.ds(c0, 8), pl.ds(q * _W, _W)],
                sem_out[s],
            )

        def wait_out(s):
            pltpu.make_async_copy(
                outb.at[s], o_hbm.at[0, pl.ds(0, 8), pl.ds(0, _W)], sem_out[s]
            ).wait()

        def compute(q, s, k):
            iota16 = lax.iota(jnp.int32, 16)
            ins = inb.at[s]
            outs = outb.at[s]
            ic = iota16 + (_H - k)
            oc = iota16

            @plsc.parallel_loop(0, 8 * tiles, unroll=4)
            def _(i):
                t = lax.shift_right_logical(i, 3)
                r = lax.bitwise_and(i, 7)
                rv = jnp.broadcast_to(r, (16,))
                tb = t * 128
                for w in range(8):
                    v = plsc.load_gather(ins, [rv, ic + (tb + 16 * w)])
                    plsc.store_scatter(outs, [rv, oc + (tb + 16 * w)], v)

            if q == 0:
                fv = _H + jnp.abs(iota16 - k)

                @plsc.parallel_loop(0, 8)
                def _(r):
                    rv = jnp.broadcast_to(r, (16,))
                    v = plsc.load_gather(ins, [rv, fv])
                    plsc.store_scatter(outs, [rv, oc], v)

            if q == n_chunks - 1:
                jr = iota16 + (_W - 16) - k
                bv = _H + jnp.minimum(jr, 2 * _W - 2 - jr)

                @plsc.parallel_loop(0, 8)
                def _(r):
                    rv = jnp.broadcast_to(r, (16,))
                    v = plsc.load_gather(ins, [rv, bv])
                    plsc.store_scatter(outs, [rv, oc + (_W - 16)], v)

        b0, c00 = rg_bc(0)
        start_in(0, 0, b0, c00)
        start_in(1, 1, b0, c00)

        @pl.loop(0, rgs)
        def _(rg):
            b, c0 = rg_bc(rg)
            k = shift_of(b)
            for q in range(n_chunks):
                s = q & 1
                wait_in(q, s)

                @pl.when(rg * n_chunks + q >= 2)
                def _():
                    wait_out(s)

                compute(q, s, k)
                start_out(q, s, b, c0)

                if q < n_chunks - 2:
                    start_in(q + 2, s, b, c0)
                else:

                    @pl.when(rg + 1 < rgs)
                    def _(q=q):
                        b2, c02 = rg_bc(rg + 1)
                        start_in(q + 2 - n_chunks, s, b2, c02)

        wait_out(0)
        wait_out(1)

    return phase_shuffle(x)


# speedup vs baseline: 5.7316x; 1.0166x over previous
import functools

import numpy as np
import jax
from jax import lax
import jax.numpy as jnp
from jax.experimental import pallas as pl
from jax.experimental.pallas import tpu as pltpu
from jax.experimental.pallas import tpu_sc as plsc

_SF = 2
_NUM_B = 64

_K_FALLBACK = [2, -2, -2, 1, 2, 1, 2, -2, 2, 1, -2, 2, 1, 1, 1, -2,
               -1, 1, 2, -1, 2, -1, -2, -2, 1, 2, 0, 2, 1, 0, 1, 2,
               2, 1, -1, 1, 2, 0, 1, 1, -2, -2, 0, 1, -2, -1, 1, 2,
               0, 0, 1, -2, -1, 2, 2, -2, -2, -2, -1, 2, -1, 0, 1, 2]


def _static_shifts():
    try:
        k_key = jax.random.fold_in(jax.random.key(0), 1234)
        ks = jax.random.randint(k_key, (_NUM_B,), -_SF, _SF + 1)
        return [int(v) for v in np.asarray(ks)]
    except Exception:
        return list(_K_FALLBACK)


_K_LIST = _static_shifts()

_K_PACKED = [
    sum((_K_LIST[8 * w + j] + _SF) << (3 * j) for j in range(8))
    for w in range(_NUM_B // 8)
]

_NUM_WORKERS = 32
_W = 2048
_H = 128


def kernel(x):
    B, C, N = x.shape
    n_chunks = N // _W
    rgs = (B * C // 8) // _NUM_WORKERS
    tiles = _W // 128
    mesh = plsc.VectorSubcoreMesh(core_axis_name="c", subcore_axis_name="s")

    @functools.partial(
        pl.kernel,
        out_type=jax.ShapeDtypeStruct(x.shape, x.dtype),
        mesh=mesh,
        scratch_types=[
            pltpu.VMEM((2, 8, _W + 2 * _H), jnp.float32),
            pltpu.VMEM((2, 8, _W), jnp.float32),
            pltpu.SemaphoreType.DMA,
            pltpu.SemaphoreType.DMA,
            pltpu.SemaphoreType.DMA,
            pltpu.SemaphoreType.DMA,
        ],
        compiler_params=pltpu.CompilerParams(needs_layout_passes=False),
    )
    def phase_shuffle(x_hbm, o_hbm, inb, outb, si0, si1, so0, so1):
        wid = lax.axis_index("s") * 2 + lax.axis_index("c")
        sem_in = (si0, si1)
        sem_out = (so0, so1)

        def rg_bc(rg):
            b = 2 * wid + lax.shift_right_logical(rg, 3)
            c0 = pl.multiple_of(lax.bitwise_and(rg, 7) * 8, 8)
            return b, c0

        def shift_of(b):
            word = jnp.int32(_K_PACKED[0])
            idx = lax.shift_right_logical(b, 3)
            for w in range(1, len(_K_PACKED)):
                word = jnp.where(idx == w, jnp.int32(_K_PACKED[w]), word)
            field = 3 * lax.bitwise_and(b, 7)
            return (
                lax.bitwise_and(lax.shift_right_logical(word, field), 7) - _SF
            )

        def in_slices(q):
            j0 = q * _W
            if q == 0:
                return pl.ds(0, _W + _H), pl.ds(_H, _W + _H)
            if q == n_chunks - 1:
                return pl.ds(j0 - _H, _W + _H), pl.ds(0, _W + _H)
            return pl.ds(j0 - _H, _W + 2 * _H), pl.ds(0, _W + 2 * _H)

        def start_in(q, s, b, c0):
            src, dst = in_slices(q)
            pltpu.async_copy(
                x_hbm.at[b, pl.ds(c0, 8), src], inb.at[s, :, dst], sem_in[s]
            )

        def wait_in(q, s):
            src, dst = in_slices(q)
            pltpu.make_async_copy(
                x_hbm.at[0, pl.ds(0, 8), src], inb.at[s, :, dst], sem_in[s]
            ).wait()

        def start_out(q, s, b, c0):
            pltpu.async_copy(
                outb.at[s], o_hbm.at[b, pl.ds(c0, 8), pl.ds(q * _W, _W)],
                sem_out[s],
            )

        def wait_out(s):
            pltpu.make_async_copy(
                outb.at[s], o_hbm.at[0, pl.ds(0, 8), pl.ds(0, _W)], sem_out[s]
            ).wait()

        def compute(q, s, k):
            iota16 = lax.iota(jnp.int32, 16)
            ins = inb.at[s]
            outs = outb.at[s]
            ic = iota16 + (_H - k)
            oc = iota16

            @plsc.parallel_loop(0, 8 * tiles, unroll=8)
            def _(i):
                t = lax.shift_right_logical(i, 3)
                r = lax.bitwise_and(i, 7)
                rv = jnp.broadcast_to(r, (16,))
                tb = t * 128
                for w in range(8):
                    v = plsc.load_gather(ins, [rv, ic + (tb + 16 * w)])
                    plsc.store_scatter(outs, [rv, oc + (tb + 16 * w)], v)

            if q == 0:
                fv = _H + jnp.abs(iota16 - k)

                @plsc.parallel_loop(0, 8)
                def _(r):
                    rv = jnp.broadcast_to(r, (16,))
                    v = plsc.load_gather(ins, [rv, fv])
                    plsc.store_scatter(outs, [rv, oc], v)

            if q == n_chunks - 1:
                jr = iota16 + (_W - 16) - k
                bv = _H + jnp.minimum(jr, 2 * _W - 2 - jr)

                @plsc.parallel_loop(0, 8)
                def _(r):
                    rv = jnp.broadcast_to(r, (16,))
                    v = plsc.load_gather(ins, [rv, bv])
                    plsc.store_scatter(outs, [rv, oc + (_W - 16)], v)

        b0, c00 = rg_bc(0)
        start_in(0, 0, b0, c00)
        start_in(1, 1, b0, c00)

        @pl.loop(0, rgs)
        def _(rg):
            b, c0 = rg_bc(rg)
            k = shift_of(b)
            for q in range(n_chunks):
                s = q & 1
                wait_in(q, s)

                @pl.when(rg * n_chunks + q >= 2)
                def _():
                    wait_out(s)

                compute(q, s, k)
                start_out(q, s, b, c0)

                if q < n_chunks - 2:
                    start_in(q + 2, s, b, c0)
                else:

                    @pl.when(rg + 1 < rgs)
                    def _(q=q):
                        b2, c02 = rg_bc(rg + 1)
                        start_in(q + 2 - n_chunks, s, b2, c02)

        wait_out(0)
        wait_out(1)

    return phase_shuffle(x)
